# baseline (speedup 1.0000x reference)
import math

import jax
import jax.numpy as jnp
from jax.experimental import pallas as pl
from jax.experimental.pallas import tpu as pltpu

B = 1
S = 2048
D = 768
R = 384
H = 12
DH = R // H
NC = 8
NE = 8
KC = 2
KE = 2

_NEG_INF = float("-inf")


def _top2_route(scores, n):
    iota = jax.lax.broadcasted_iota(jnp.int32, scores.shape, 1)
    v0 = jnp.max(scores, axis=-1, keepdims=True)
    i0 = jnp.min(jnp.where(scores == v0, iota, n), axis=-1, keepdims=True)
    masked = jnp.where(iota == i0, _NEG_INF, scores)
    v1 = jnp.max(masked, axis=-1, keepdims=True)
    i1 = jnp.min(jnp.where(masked == v1, iota, n), axis=-1, keepdims=True)
    e = jnp.exp(v1 - v0)
    denom = 1.0 + e
    w0 = 1.0 / denom
    w1 = e / denom
    w_pair = jnp.concatenate([w0, w1], axis=1)
    i_pair = jnp.concatenate([i0, i1], axis=1)
    w_full = (jnp.where(iota == i0, w0, 0.0)
              + jnp.where(iota == i1, w1, 0.0))
    return w_pair, i_pair, w_full


def _qkv_kernel(x_ref, sq_ref, sk_ref, sv_ref, cn_ref,
                q_ref, k_ref, v_ref,
                qw_ref, qi_ref, kw_ref, ki_ref, vw_ref, vi_ref):
    x = x_ref[...]
    qw, qi, qf = _top2_route(sq_ref[...], NC)
    kw, ki, kf = _top2_route(sk_ref[...], NC)
    vw, vi, vf = _top2_route(sv_ref[...], NC)
    qw_ref[...] = qw
    qi_ref[...] = qi
    kw_ref[...] = kw
    ki_ref[...] = ki
    vw_ref[...] = vw
    vi_ref[...] = vi
    bs = x.shape[0]
    acc_q = jnp.zeros((bs, R), dtype=jnp.float32)
    acc_k = jnp.zeros((bs, R), dtype=jnp.float32)
    acc_v = jnp.zeros((bs, R), dtype=jnp.float32)
    for nidx in range(NC):
        p = jnp.dot(x, cn_ref[nidx], preferred_element_type=jnp.float32)
        acc_q = acc_q + qf[:, nidx:nidx + 1] * p
        acc_k = acc_k + kf[:, nidx:nidx + 1] * p
        acc_v = acc_v + vf[:, nidx:nidx + 1] * p
    q_ref[...] = acc_q
    k_ref[...] = acc_k
    v_ref[...] = acc_v


def _expand_kernel(a_ref, so_ref, en_ref, out_ref, ow_ref, oi_ref):
    a = a_ref[...]
    ow, oi, of = _top2_route(so_ref[...], NE)
    ow_ref[...] = ow
    oi_ref[...] = oi
    bs = a.shape[0]
    acc = jnp.zeros((bs, D), dtype=jnp.float32)
    for nidx in range(NE):
        p = jnp.dot(a, en_ref[nidx], preferred_element_type=jnp.float32)
        acc = acc + of[:, nidx:nidx + 1] * p
    out_ref[...] = acc


def _stage1(x2d, sq, sk, sv, compress_neurons):
    bs = 512
    grid = (S // bs,)
    return pl.pallas_call(
        _qkv_kernel,
        grid=grid,
        in_specs=[
            pl.BlockSpec((bs, D), lambda i: (i, 0)),
            pl.BlockSpec((bs, NC), lambda i: (i, 0)),
            pl.BlockSpec((bs, NC), lambda i: (i, 0)),
            pl.BlockSpec((bs, NC), lambda i: (i, 0)),
            pl.BlockSpec((NC, D, R), lambda i: (0, 0, 0)),
        ],
        out_specs=[
            pl.BlockSpec((bs, R), lambda i: (i, 0)),
            pl.BlockSpec((bs, R), lambda i: (i, 0)),
            pl.BlockSpec((bs, R), lambda i: (i, 0)),
            pl.BlockSpec((bs, KC), lambda i: (i, 0)),
            pl.BlockSpec((bs, KC), lambda i: (i, 0)),
            pl.BlockSpec((bs, KC), lambda i: (i, 0)),
            pl.BlockSpec((bs, KC), lambda i: (i, 0)),
            pl.BlockSpec((bs, KC), lambda i: (i, 0)),
            pl.BlockSpec((bs, KC), lambda i: (i, 0)),
        ],
        out_shape=[
            jax.ShapeDtypeStruct((S, R), jnp.float32),
            jax.ShapeDtypeStruct((S, R), jnp.float32),
            jax.ShapeDtypeStruct((S, R), jnp.float32),
            jax.ShapeDtypeStruct((S, KC), jnp.float32),
            jax.ShapeDtypeStruct((S, KC), jnp.int32),
            jax.ShapeDtypeStruct((S, KC), jnp.float32),
            jax.ShapeDtypeStruct((S, KC), jnp.int32),
            jax.ShapeDtypeStruct((S, KC), jnp.float32),
            jax.ShapeDtypeStruct((S, KC), jnp.int32),
        ],
        compiler_params=pltpu.CompilerParams(
            dimension_semantics=("parallel",)),
    )(x2d, sq, sk, sv, compress_neurons)


def _stage3(attn2d, so, expand_neurons):
    bs = 512
    grid = (S // bs,)
    return pl.pallas_call(
        _expand_kernel,
        grid=grid,
        in_specs=[
            pl.BlockSpec((bs, R), lambda i: (i, 0)),
            pl.BlockSpec((bs, NE), lambda i: (i, 0)),
            pl.BlockSpec((NE, R, D), lambda i: (0, 0, 0)),
        ],
        out_specs=[
            pl.BlockSpec((bs, D), lambda i: (i, 0)),
            pl.BlockSpec((bs, KE), lambda i: (i, 0)),
            pl.BlockSpec((bs, KE), lambda i: (i, 0)),
        ],
        out_shape=[
            jax.ShapeDtypeStruct((S, D), jnp.float32),
            jax.ShapeDtypeStruct((S, KE), jnp.float32),
            jax.ShapeDtypeStruct((S, KE), jnp.int32),
        ],
        compiler_params=pltpu.CompilerParams(
            dimension_semantics=("parallel",)),
    )(attn2d, so, expand_neurons)


@jax.jit
def kernel(x, compress_neurons, expand_neurons, Wq, Wk, Wv, Wo):
    sq = (x @ Wq.T).reshape(S, NC)
    sk = (x @ Wk.T).reshape(S, NC)
    sv = (x @ Wv.T).reshape(S, NC)
    sq, sk, sv = jax.lax.optimization_barrier((sq, sk, sv))

    x2d = x.reshape(S, D)
    q2d, k2d, v2d, qw, qi, kw, ki, vw, vi = _stage1(
        x2d, sq, sk, sv, compress_neurons)

    Qh = q2d.reshape(B, S, H, DH).transpose(0, 2, 1, 3)
    Kh = k2d.reshape(B, S, H, DH).transpose(0, 2, 1, 3)
    Vh = v2d.reshape(B, S, H, DH).transpose(0, 2, 1, 3)
    attn_scores = jnp.einsum('bhqd,bhkd->bhqk', Qh, Kh) / math.sqrt(DH)
    attn = jax.nn.softmax(attn_scores, axis=-1)
    attn_out = jnp.einsum('bhqk,bhkd->bhqd', attn, Vh)
    attn2d = attn_out.transpose(0, 2, 1, 3).reshape(S, R)

    so = (attn2d.reshape(B, S, R) @ Wo.T).reshape(S, NE)
    so = jax.lax.optimization_barrier(so)
    out2d, ow, oi = _stage3(attn2d, so, expand_neurons)

    r3 = lambda a: a.reshape(B, S, a.shape[-1])
    return (r3(out2d), r3(qw), r3(qi), r3(kw), r3(ki),
            r3(vw), r3(vi), r3(ow), r3(oi))
